# baseline (speedup 1.0000x reference)
import jax
import jax.numpy as jnp
from jax import lax
from jax.experimental import pallas as pl
from jax.experimental.pallas import tpu as pltpu
from jax.experimental.pallas import tpu_sc as plsc

N = 10000
E = 160000
NODE_DIM = 128
EDGE_DIM = 16
ATOM = 32
CONV = 128
NGRAPH = 64

NC = 2
NS = 16
NW = NC * NS
CHUNK = 128
N_PAD = 10240
E_PAD = 163840
BPW = E_PAD // NW
CPW = BPW // CHUNK
RPT = N_PAD // NS

_SC_MESH = dict(core_axis_name="c", subcore_axis_name="s",
                num_cores=NC, num_subcores=NS)


NB = 4
GQ = 4
CPQ = CPW // GQ
QROWS = CPQ * CHUNK


def _sc_gather_body(tab_hbm, idx_hbm, out_hbm, idx_all, rows, *sems):
    gs, ws = sems[:2], sems[2:]
    c = lax.axis_index("c")
    s = lax.axis_index("s")
    wid = c * NS + s
    base = wid * BPW
    pltpu.sync_copy(idx_hbm.at[wid], idx_all)
    gd = [[None] * CPQ for _ in range(2)]
    wd = [None, None]

    def issue(qtr):
        b = qtr % 2
        for j in range(CPQ):
            jj = qtr * CPQ + j
            gd[b][j] = pltpu.async_copy(
                tab_hbm.at[idx_all.at[jj]],
                rows.at[b, pl.ds(j * CHUNK, CHUNK)], gs[b])

    issue(0)
    for qtr in range(GQ):
        b = qtr % 2
        if qtr + 1 < GQ:
            if qtr >= 1:
                wd[(qtr + 1) % 2].wait()
            issue(qtr + 1)
        for j in range(CPQ):
            gd[b][j].wait()
        wd[b] = pltpu.async_copy(
            rows.at[b], out_hbm.at[pl.ds(base + qtr * QROWS, QROWS)], ws[b])
    wd[0].wait()
    wd[1].wait()


def _sc_gather(tab, idx3):
    return pl.kernel(
        _sc_gather_body,
        out_type=jax.ShapeDtypeStruct((E_PAD, ATOM), jnp.float32),
        compiler_params=pltpu.CompilerParams(use_tc_tiling_on_sc=False),
        mesh=plsc.VectorSubcoreMesh(**_SC_MESH),
        scratch_types=[
            pltpu.VMEM((CPW, CHUNK), jnp.int32),
            pltpu.VMEM((2, QROWS, ATOM), jnp.float32),
        ] + [pltpu.SemaphoreType.DMA] * 4,
    )(tab, idx3)


def _make_sc_scatter(width):
    def body(vals_hbm, idx_hbm, zeros_hbm, out_hbm, idx_all, vals, acc, *sems):
        ls, ss = sems[:NB], sems[NB:]
        c = lax.axis_index("c")
        s = lax.axis_index("s")
        wid = c * NS + s
        base = wid * BPW
        pltpu.sync_copy(zeros_hbm.at[pl.ds(s * RPT, RPT)],
                        acc.at[pl.ds(s * RPT, RPT)])
        pltpu.sync_copy(idx_hbm.at[wid], idx_all)
        plsc.subcore_barrier()
        ld = [None] * CPW
        sd = [None] * CPW

        def start(j):
            b = j % NB
            ld[j] = pltpu.async_copy(
                vals_hbm.at[pl.ds(base + j * CHUNK, CHUNK)], vals.at[b], ls[b])

        start(0)
        for j in range(CPW):
            b = j % NB
            if j + 1 < CPW:
                if j + 1 >= NB:
                    sd[j + 1 - NB].wait()
                start(j + 1)
            ld[j].wait()
            sd[j] = pltpu.async_copy(vals.at[b], acc.at[idx_all.at[j]],
                                     ss[b], add=True)
        for j in range(CPW - NB, CPW):
            sd[j].wait()
        plsc.subcore_barrier()
        pltpu.sync_copy(acc.at[pl.ds(s * RPT, RPT)],
                        out_hbm.at[c, pl.ds(s * RPT, RPT)])

    def run(vals, idx3, zeros):
        return pl.kernel(
            body,
            out_type=jax.ShapeDtypeStruct((NC, N_PAD, width), jnp.float32),
            compiler_params=pltpu.CompilerParams(use_tc_tiling_on_sc=False),
            mesh=plsc.VectorSubcoreMesh(**_SC_MESH),
            scratch_types=[
                pltpu.VMEM((CPW, CHUNK), jnp.int32),
                pltpu.VMEM((NB, CHUNK, width), jnp.float32),
                pltpu.VMEM_SHARED((N_PAD, width), jnp.float32),
            ] + [pltpu.SemaphoreType.DMA] * (2 * NB),
        )(vals, idx3, zeros)

    return run


_sc_scatter32 = _make_sc_scatter(ATOM)
_sc_scatter48 = _make_sc_scatter(48)


def _linrelu_body(x_ref, w_ref, b_ref, o_ref):
    o_ref[...] = jax.nn.relu(
        jnp.dot(x_ref[...], w_ref[...], preferred_element_type=jnp.float32)
        + b_ref[...]).astype(o_ref.dtype)


def _linrelu(x, w, b, tr, out_dtype=jnp.float32):
    rows, k = x.shape
    d = w.shape[1]
    return pl.pallas_call(
        _linrelu_body,
        grid=(rows // tr,),
        in_specs=[
            pl.BlockSpec((tr, k), lambda i: (i, 0)),
            pl.BlockSpec((k, d), lambda i: (0, 0)),
            pl.BlockSpec((1, d), lambda i: (0, 0)),
        ],
        out_specs=pl.BlockSpec((tr, d), lambda i: (i, 0)),
        out_shape=jax.ShapeDtypeStruct((rows, d), out_dtype),
    )(x, w, b)


TE = 1024


def _msg_body(hid_ref, xs_ref, n2wT_ref, r_ref, n2bm_ref, o_ref):
    hid = hid_ref[...]
    xs = xs_ref[...]
    xsb = xs.astype(jnp.bfloat16)
    q = None
    for j in range(8):
        wj = jnp.dot(hid, n2wT_ref[:, 128 * j:128 * (j + 1)],
                     preferred_element_type=jnp.float32)
        xej = jnp.dot(xsb, r_ref[:, 128 * j:128 * (j + 1)],
                      preferred_element_type=jnp.float32)
        pj = wj * xej
        q = pj if q is None else q + pj
    msg = (lax.slice(q, (0, 0), (TE, 32)) + lax.slice(q, (0, 32), (TE, 64))
           + lax.slice(q, (0, 64), (TE, 96))
           + lax.slice(q, (0, 96), (TE, 128)))
    msg = msg + jnp.dot(xs, n2bm_ref[...], preferred_element_type=jnp.float32)
    if o_ref.shape[1] == ATOM + 16:
        msg = jnp.concatenate(
            [msg, jnp.ones((TE, 16), jnp.float32)], axis=1)
    o_ref[...] = msg


def _tc_msg(hidden, xs, n2wT, rmat, n2bm, width=ATOM):
    aa = ATOM * ATOM
    return pl.pallas_call(
        _msg_body,
        grid=(E_PAD // TE,),
        in_specs=[
            pl.BlockSpec((TE, CONV), lambda i: (i, 0)),
            pl.BlockSpec((TE, ATOM), lambda i: (i, 0)),
            pl.BlockSpec((CONV, aa), lambda i: (0, 0)),
            pl.BlockSpec((ATOM, aa), lambda i: (0, 0)),
            pl.BlockSpec((ATOM, ATOM), lambda i: (0, 0)),
        ],
        out_specs=pl.BlockSpec((TE, width), lambda i: (i, 0)),
        out_shape=jax.ShapeDtypeStruct((E_PAD, width), jnp.float32),
    )(hidden, xs, n2wT, rmat, n2bm)


TN = 2048


def _gru_cell_tc(p_ref, deg, h, gw_refs):
    wir, wiz, win, whr, whz, whn, brz_r, brz_z, bin_, bhn = gw_refs
    agg = p_ref[0, :, 0:ATOM] + p_ref[1, :, 0:ATOM]
    m = jax.nn.relu(agg / deg)

    def mm(a, b):
        return jnp.dot(a, b[...], preferred_element_type=jnp.float32)

    r = jax.nn.sigmoid(mm(m, wir) + mm(h, whr) + brz_r[...])
    z = jax.nn.sigmoid(mm(m, wiz) + mm(h, whz) + brz_z[...])
    n = jnp.tanh(mm(m, win) + bin_[...] + r * (mm(h, whn) + bhn[...]))
    return (1.0 - z) * n + z * h


def _gru_body(p_ref, h_ref, wir, wiz, win, whr, whz, whn,
              brz_r, brz_z, bin_, bhn, o_ref):
    deg = jnp.maximum(p_ref[0, :, ATOM:ATOM + 1] + p_ref[1, :, ATOM:ATOM + 1],
                      1.0)
    o_ref[...] = _gru_cell_tc(
        p_ref, deg, h_ref[...],
        (wir, wiz, win, whr, whz, whn, brz_r, brz_z, bin_, bhn))


def _tc_gru(parts48, h, gw):
    full = lambda shape: pl.BlockSpec(shape, lambda i: tuple(0 for _ in shape))
    return pl.pallas_call(
        _gru_body,
        grid=(N_PAD // TN,),
        in_specs=[
            pl.BlockSpec((NC, TN, ATOM + 16), lambda i: (0, i, 0)),
            pl.BlockSpec((TN, ATOM), lambda i: (i, 0)),
        ] + [full((ATOM, ATOM))] * 6 + [full((1, ATOM))] * 4,
        out_specs=pl.BlockSpec((TN, ATOM), lambda i: (i, 0)),
        out_shape=jax.ShapeDtypeStruct((N_PAD, ATOM), jnp.float32),
    )(parts48, h, *gw)


EMB_STEPS = 3


def _tail_body(p32_ref, p48_ref, h_ref, gwir, gwiz, gwin, gwhr, gwhz, gwhn,
               gbrz_r, gbrz_z, gbin, gbhn,
               batch_ref, batchT_ref, wi, wf, wg, wo, ui, uf, ug, uo,
               bi, bf, bg_, bo, bng, bnb, bnrm, bnrv, m1w, m1b, m2w, m2b,
               pw, pb, o_ref):
    deg = jnp.maximum(
        p48_ref[0, :, ATOM:ATOM + 1] + p48_ref[1, :, ATOM:ATOM + 1], 1.0)
    h2 = _gru_cell_tc(
        p32_ref, deg, h_ref[...],
        (gwir, gwiz, gwin, gwhr, gwhz, gwhn, gbrz_r, gbrz_z, gbin, gbhn))
    xhT = h2.T
    batch = batch_ref[...]
    batchT = batchT_ref[...]
    gids = lax.broadcasted_iota(jnp.int32, (NGRAPH, N_PAD), 0)
    gidsT = lax.broadcasted_iota(jnp.int32, (N_PAD, NGRAPH), 1)
    mb = gids == batch
    mf = mb.astype(jnp.float32)
    mfT = (gidsT == batchT).astype(jnp.float32)

    def mm(a, b):
        return jnp.dot(a, b, preferred_element_type=jnp.float32)

    qsT = jnp.zeros((2 * ATOM, NGRAPH), jnp.float32)
    hsT = jnp.zeros((ATOM, NGRAPH), jnp.float32)
    csT = jnp.zeros((ATOM, NGRAPH), jnp.float32)
    for _ in range(EMB_STEPS):
        ig = jax.nn.sigmoid(mm(wi[...], qsT) + mm(ui[...], hsT) + bi[...])
        fg = jax.nn.sigmoid(mm(wf[...], qsT) + mm(uf[...], hsT) + bf[...])
        gg = jnp.tanh(mm(wg[...], qsT) + mm(ug[...], hsT) + bg_[...])
        og = jax.nn.sigmoid(mm(wo[...], qsT) + mm(uo[...], hsT) + bo[...])
        csT = fg * csT + ig * gg
        hsT = og * jnp.tanh(csT)
        qT = hsT
        qbT = mm(qT, mf)
        e = jnp.sum(xhT * qbT, axis=0, keepdims=True)
        e_b = jnp.broadcast_to(e, (NGRAPH, N_PAD))
        mmax = jnp.max(jnp.where(mb, e_b, -1e30), axis=1, keepdims=True)
        mmax_n = jnp.sum(mf * mmax, axis=0, keepdims=True)
        a = jnp.exp(e - mmax_n)
        denom = jnp.sum(mf * a, axis=1, keepdims=True)
        denom_n = jnp.sum(mf * denom, axis=0, keepdims=True)
        anorm = jnp.where(denom_n > 0.0,
                          a / jnp.maximum(denom_n, 1e-30), 0.0)
        rT = mm(xhT * anorm, mfT)
        qsT = jnp.concatenate([qT, rT], axis=0)

    o = (qsT - bnrm[...]) / jnp.sqrt(bnrv[...] + 1e-5) * bng[...] + bnb[...]
    o1 = jax.nn.relu(mm(m1w[...], o) + m1b[...])
    o2 = jax.nn.relu(mm(m2w[...], o1) + m2b[...])
    o_ref[...] = mm(pw[...], o2) + pb[...]


def _tc_tail(parts32, parts48, h, gw, batch2d, batchT, weights):
    return pl.pallas_call(
        _tail_body,
        out_shape=jax.ShapeDtypeStruct((1, NGRAPH), jnp.float32),
    )(parts32, parts48, h, *gw, batch2d, batchT, *weights)


def kernel(x, edge_attr, edge_index, batch, lin_w, lin_b, n1_w, n1_b, n2_w,
           n2_b, gru_wih, gru_whh, gru_bih, gru_bhh, lstm_wih, lstm_whh,
           lstm_bih, lstm_bhh, bn_g, bn_b, bn_rm, bn_rv, m1_w, m1_b, m2_w,
           m2_b, p_w, p_b):
    f32 = jnp.float32
    src = edge_index[0]
    dst = edge_index[1]
    pad_e = E_PAD - E
    src3 = jnp.concatenate([src, jnp.zeros((pad_e,), jnp.int32)]
                           ).reshape(NW, CPW, CHUNK)
    dst3 = jnp.concatenate([dst, jnp.full((pad_e,), N, jnp.int32)]
                           ).reshape(NW, CPW, CHUNK)
    x_p = jnp.pad(x, ((0, N_PAD - N), (0, 0)))
    ea_p = jnp.pad(edge_attr, ((0, pad_e), (0, 0)))
    zeros32 = jnp.zeros((N_PAD, ATOM), f32)
    zeros48 = jnp.zeros((N_PAD, 48), f32)
    bf16 = jnp.bfloat16
    rmat = jnp.kron(jnp.eye(ATOM, dtype=bf16), jnp.ones((1, ATOM), bf16))
    n2bm = n2_b.reshape(ATOM, ATOM)
    n2wT = n2_w.T.astype(bf16)

    def g3(w):
        return w[0:ATOM].T, w[ATOM:2 * ATOM].T, w[2 * ATOM:3 * ATOM].T

    wir, wiz, win = g3(gru_wih)
    whr, whz, whn = g3(gru_whh)
    row = lambda v: v.reshape(1, -1)
    gw = (wir, wiz, win, whr, whz, whn,
          row(gru_bih[0:ATOM] + gru_bhh[0:ATOM]),
          row(gru_bih[ATOM:2 * ATOM] + gru_bhh[ATOM:2 * ATOM]),
          row(gru_bih[2 * ATOM:]), row(gru_bhh[2 * ATOM:]))

    def g4(w):
        return tuple(w[k * ATOM:(k + 1) * ATOM] for k in range(4))

    wi, wf, wg, wo = g4(lstm_wih)
    ui, uf, ug, uo = g4(lstm_whh)
    col = lambda v: v.reshape(-1, 1)
    lb = tuple(col(lstm_bih[k * ATOM:(k + 1) * ATOM]
                   + lstm_bhh[k * ATOM:(k + 1) * ATOM]) for k in range(4))
    tail_w = (wi, wf, wg, wo, ui, uf, ug, uo) + lb + (
        col(bn_g), col(bn_b), col(bn_rm), col(bn_rv),
        m1_w, col(m1_b), m2_w, col(m2_b), p_w, col(p_b))

    batch2d = jnp.pad(batch, (0, N_PAD - N),
                      constant_values=NGRAPH).reshape(1, N_PAD)
    batchT = batch2d.reshape(N_PAD, 1)

    xh = _linrelu(x_p, lin_w.T, lin_b.reshape(1, -1), 2048)
    hidden = _linrelu(ea_p, n1_w.T, n1_b.reshape(1, -1), 2048,
                      out_dtype=bf16)

    xs = _sc_gather(xh, src3)
    msg48 = _tc_msg(hidden, xs, n2wT, rmat, n2bm, width=ATOM + 16)
    parts48 = _sc_scatter48(msg48, dst3, zeros48)
    h1 = _tc_gru(parts48, xh, gw)

    xs2 = _sc_gather(h1, src3)
    msg2 = _tc_msg(hidden, xs2, n2wT, rmat, n2bm)
    parts32 = _sc_scatter32(msg2, dst3, zeros32)
    out_t = _tc_tail(parts32, parts48, h1, gw, batch2d, batchT, tail_w)
    return out_t.reshape(NGRAPH, 1)


# speedup vs baseline: 1.0347x; 1.0347x over previous
import jax
import jax.numpy as jnp
from jax import lax
from jax.experimental import pallas as pl
from jax.experimental.pallas import tpu as pltpu
from jax.experimental.pallas import tpu_sc as plsc

N = 10000
E = 160000
NODE_DIM = 128
EDGE_DIM = 16
ATOM = 32
CONV = 128
NGRAPH = 64

NC = 2
NS = 16
NW = NC * NS
CHUNK = 128
N_PAD = 10240
E_PAD = 163840
BPW = E_PAD // NW
CPW = BPW // CHUNK
RPT = N_PAD // NS

_SC_MESH = dict(core_axis_name="c", subcore_axis_name="s",
                num_cores=NC, num_subcores=NS)


NB = 4
GQ = 4
CPQ = CPW // GQ
QROWS = CPQ * CHUNK


def _sc_gather_body(tab_hbm, idx_hbm, out_hbm, idx_all, rows, *sems):
    gs, ws = sems[:2], sems[2:]
    c = lax.axis_index("c")
    s = lax.axis_index("s")
    wid = c * NS + s
    base = wid * BPW
    pltpu.sync_copy(idx_hbm.at[wid], idx_all)
    gd = [[None] * CPQ for _ in range(2)]
    wd = [None, None]

    def issue(qtr):
        b = qtr % 2
        for j in range(CPQ):
            jj = qtr * CPQ + j
            gd[b][j] = pltpu.async_copy(
                tab_hbm.at[idx_all.at[jj]],
                rows.at[b, pl.ds(j * CHUNK, CHUNK)], gs[b])

    issue(0)
    for qtr in range(GQ):
        b = qtr % 2
        if qtr + 1 < GQ:
            if qtr >= 1:
                wd[(qtr + 1) % 2].wait()
            issue(qtr + 1)
        for j in range(CPQ):
            gd[b][j].wait()
        wd[b] = pltpu.async_copy(
            rows.at[b], out_hbm.at[pl.ds(base + qtr * QROWS, QROWS)], ws[b])
    wd[0].wait()
    wd[1].wait()


def _sc_gather(tab, idx3):
    return pl.kernel(
        _sc_gather_body,
        out_type=jax.ShapeDtypeStruct((E_PAD, ATOM), jnp.float32),
        compiler_params=pltpu.CompilerParams(use_tc_tiling_on_sc=False),
        mesh=plsc.VectorSubcoreMesh(**_SC_MESH),
        scratch_types=[
            pltpu.VMEM((CPW, CHUNK), jnp.int32),
            pltpu.VMEM((2, QROWS, ATOM), jnp.float32),
        ] + [pltpu.SemaphoreType.DMA] * 4,
    )(tab, idx3)


def _make_sc_scatter(width):
    def body(vals_hbm, idx_hbm, zeros_hbm, out_hbm, idx_all, vals, acc, *sems):
        ls, ss = sems[:NB], sems[NB:]
        c = lax.axis_index("c")
        s = lax.axis_index("s")
        wid = c * NS + s
        base = wid * BPW
        pltpu.sync_copy(zeros_hbm.at[pl.ds(s * RPT, RPT)],
                        acc.at[pl.ds(s * RPT, RPT)])
        pltpu.sync_copy(idx_hbm.at[wid], idx_all)
        plsc.subcore_barrier()
        ld = [None] * CPW
        sd = [None] * CPW

        def start(j):
            b = j % NB
            ld[j] = pltpu.async_copy(
                vals_hbm.at[pl.ds(base + j * CHUNK, CHUNK)], vals.at[b], ls[b])

        start(0)
        for j in range(CPW):
            b = j % NB
            if j + 1 < CPW:
                if j + 1 >= NB:
                    sd[j + 1 - NB].wait()
                start(j + 1)
            ld[j].wait()
            sd[j] = pltpu.async_copy(vals.at[b], acc.at[idx_all.at[j]],
                                     ss[b], add=True)
        for j in range(CPW - NB, CPW):
            sd[j].wait()
        plsc.subcore_barrier()
        pltpu.sync_copy(acc.at[pl.ds(s * RPT, RPT)],
                        out_hbm.at[c, pl.ds(s * RPT, RPT)])

    def run(vals, idx3, zeros):
        return pl.kernel(
            body,
            out_type=jax.ShapeDtypeStruct((NC, N_PAD, width), jnp.float32),
            compiler_params=pltpu.CompilerParams(use_tc_tiling_on_sc=False),
            mesh=plsc.VectorSubcoreMesh(**_SC_MESH),
            scratch_types=[
                pltpu.VMEM((CPW, CHUNK), jnp.int32),
                pltpu.VMEM((NB, CHUNK, width), jnp.float32),
                pltpu.VMEM_SHARED((N_PAD, width), jnp.float32),
            ] + [pltpu.SemaphoreType.DMA] * (2 * NB),
        )(vals, idx3, zeros)

    return run


_sc_scatter32 = _make_sc_scatter(ATOM)
_sc_scatter48 = _make_sc_scatter(48)


def _linrelu_body(x_ref, w_ref, b_ref, o_ref):
    o_ref[...] = jax.nn.relu(
        jnp.dot(x_ref[...], w_ref[...], preferred_element_type=jnp.float32)
        + b_ref[...]).astype(o_ref.dtype)


def _linrelu(x, w, b, tr, out_dtype=jnp.float32):
    rows, k = x.shape
    d = w.shape[1]
    return pl.pallas_call(
        _linrelu_body,
        grid=(rows // tr,),
        in_specs=[
            pl.BlockSpec((tr, k), lambda i: (i, 0)),
            pl.BlockSpec((k, d), lambda i: (0, 0)),
            pl.BlockSpec((1, d), lambda i: (0, 0)),
        ],
        out_specs=pl.BlockSpec((tr, d), lambda i: (i, 0)),
        out_shape=jax.ShapeDtypeStruct((rows, d), out_dtype),
    )(x, w, b)


TE = 2048


def _msg_body(hid_ref, xs_ref, n2wT_ref, r_ref, n2bm_ref, o_ref):
    hid = hid_ref[...]
    xs = xs_ref[...]
    xsb = xs.astype(jnp.bfloat16)
    q = None
    for j in range(8):
        wj = jnp.dot(hid, n2wT_ref[:, 128 * j:128 * (j + 1)],
                     preferred_element_type=jnp.float32)
        xej = jnp.dot(xsb, r_ref[:, 128 * j:128 * (j + 1)],
                      preferred_element_type=jnp.float32)
        pj = wj * xej
        q = pj if q is None else q + pj
    msg = (lax.slice(q, (0, 0), (TE, 32)) + lax.slice(q, (0, 32), (TE, 64))
           + lax.slice(q, (0, 64), (TE, 96))
           + lax.slice(q, (0, 96), (TE, 128)))
    msg = msg + jnp.dot(xs, n2bm_ref[...], preferred_element_type=jnp.float32)
    if o_ref.shape[1] == ATOM + 16:
        msg = jnp.concatenate(
            [msg, jnp.ones((TE, 16), jnp.float32)], axis=1)
    o_ref[...] = msg


def _tc_msg(hidden, xs, n2wT, rmat, n2bm, width=ATOM):
    aa = ATOM * ATOM
    return pl.pallas_call(
        _msg_body,
        grid=(E_PAD // TE,),
        in_specs=[
            pl.BlockSpec((TE, CONV), lambda i: (i, 0)),
            pl.BlockSpec((TE, ATOM), lambda i: (i, 0)),
            pl.BlockSpec((CONV, aa), lambda i: (0, 0)),
            pl.BlockSpec((ATOM, aa), lambda i: (0, 0)),
            pl.BlockSpec((ATOM, ATOM), lambda i: (0, 0)),
        ],
        out_specs=pl.BlockSpec((TE, width), lambda i: (i, 0)),
        out_shape=jax.ShapeDtypeStruct((E_PAD, width), jnp.float32),
    )(hidden, xs, n2wT, rmat, n2bm)


TN = 2048


def _gru_cell_tc(p_ref, deg, h, gw_refs):
    wir, wiz, win, whr, whz, whn, brz_r, brz_z, bin_, bhn = gw_refs
    agg = p_ref[0, :, 0:ATOM] + p_ref[1, :, 0:ATOM]
    m = jax.nn.relu(agg / deg)

    def mm(a, b):
        return jnp.dot(a, b[...], preferred_element_type=jnp.float32)

    r = jax.nn.sigmoid(mm(m, wir) + mm(h, whr) + brz_r[...])
    z = jax.nn.sigmoid(mm(m, wiz) + mm(h, whz) + brz_z[...])
    n = jnp.tanh(mm(m, win) + bin_[...] + r * (mm(h, whn) + bhn[...]))
    return (1.0 - z) * n + z * h


def _gru_body(p_ref, h_ref, wir, wiz, win, whr, whz, whn,
              brz_r, brz_z, bin_, bhn, o_ref):
    deg = jnp.maximum(p_ref[0, :, ATOM:ATOM + 1] + p_ref[1, :, ATOM:ATOM + 1],
                      1.0)
    o_ref[...] = _gru_cell_tc(
        p_ref, deg, h_ref[...],
        (wir, wiz, win, whr, whz, whn, brz_r, brz_z, bin_, bhn))


def _tc_gru(parts48, h, gw):
    full = lambda shape: pl.BlockSpec(shape, lambda i: tuple(0 for _ in shape))
    return pl.pallas_call(
        _gru_body,
        grid=(N_PAD // TN,),
        in_specs=[
            pl.BlockSpec((NC, TN, ATOM + 16), lambda i: (0, i, 0)),
            pl.BlockSpec((TN, ATOM), lambda i: (i, 0)),
        ] + [full((ATOM, ATOM))] * 6 + [full((1, ATOM))] * 4,
        out_specs=pl.BlockSpec((TN, ATOM), lambda i: (i, 0)),
        out_shape=jax.ShapeDtypeStruct((N_PAD, ATOM), jnp.float32),
    )(parts48, h, *gw)


EMB_STEPS = 3


def _tail_body(p32_ref, p48_ref, h_ref, gwir, gwiz, gwin, gwhr, gwhz, gwhn,
               gbrz_r, gbrz_z, gbin, gbhn,
               batch_ref, batchT_ref, wi, wf, wg, wo, ui, uf, ug, uo,
               bi, bf, bg_, bo, bng, bnb, bnrm, bnrv, m1w, m1b, m2w, m2b,
               pw, pb, o_ref):
    deg = jnp.maximum(
        p48_ref[0, :, ATOM:ATOM + 1] + p48_ref[1, :, ATOM:ATOM + 1], 1.0)
    h2 = _gru_cell_tc(
        p32_ref, deg, h_ref[...],
        (gwir, gwiz, gwin, gwhr, gwhz, gwhn, gbrz_r, gbrz_z, gbin, gbhn))
    xhT = h2.T
    batch = batch_ref[...]
    batchT = batchT_ref[...]
    gids = lax.broadcasted_iota(jnp.int32, (NGRAPH, N_PAD), 0)
    gidsT = lax.broadcasted_iota(jnp.int32, (N_PAD, NGRAPH), 1)
    mb = gids == batch
    mf = mb.astype(jnp.float32)
    mfT = (gidsT == batchT).astype(jnp.float32)

    def mm(a, b):
        return jnp.dot(a, b, preferred_element_type=jnp.float32)

    qsT = jnp.zeros((2 * ATOM, NGRAPH), jnp.float32)
    hsT = jnp.zeros((ATOM, NGRAPH), jnp.float32)
    csT = jnp.zeros((ATOM, NGRAPH), jnp.float32)
    for _ in range(EMB_STEPS):
        ig = jax.nn.sigmoid(mm(wi[...], qsT) + mm(ui[...], hsT) + bi[...])
        fg = jax.nn.sigmoid(mm(wf[...], qsT) + mm(uf[...], hsT) + bf[...])
        gg = jnp.tanh(mm(wg[...], qsT) + mm(ug[...], hsT) + bg_[...])
        og = jax.nn.sigmoid(mm(wo[...], qsT) + mm(uo[...], hsT) + bo[...])
        csT = fg * csT + ig * gg
        hsT = og * jnp.tanh(csT)
        qT = hsT
        qbT = mm(qT, mf)
        e = jnp.sum(xhT * qbT, axis=0, keepdims=True)
        e_b = jnp.broadcast_to(e, (NGRAPH, N_PAD))
        mmax = jnp.max(jnp.where(mb, e_b, -1e30), axis=1, keepdims=True)
        mmax_n = jnp.sum(mf * mmax, axis=0, keepdims=True)
        a = jnp.exp(e - mmax_n)
        denom = jnp.sum(mf * a, axis=1, keepdims=True)
        denom_n = jnp.sum(mf * denom, axis=0, keepdims=True)
        anorm = jnp.where(denom_n > 0.0,
                          a / jnp.maximum(denom_n, 1e-30), 0.0)
        rT = mm(xhT * anorm, mfT)
        qsT = jnp.concatenate([qT, rT], axis=0)

    o = (qsT - bnrm[...]) / jnp.sqrt(bnrv[...] + 1e-5) * bng[...] + bnb[...]
    o1 = jax.nn.relu(mm(m1w[...], o) + m1b[...])
    o2 = jax.nn.relu(mm(m2w[...], o1) + m2b[...])
    o_ref[...] = mm(pw[...], o2) + pb[...]


def _tc_tail(parts32, parts48, h, gw, batch2d, batchT, weights):
    return pl.pallas_call(
        _tail_body,
        out_shape=jax.ShapeDtypeStruct((1, NGRAPH), jnp.float32),
    )(parts32, parts48, h, *gw, batch2d, batchT, *weights)


def kernel(x, edge_attr, edge_index, batch, lin_w, lin_b, n1_w, n1_b, n2_w,
           n2_b, gru_wih, gru_whh, gru_bih, gru_bhh, lstm_wih, lstm_whh,
           lstm_bih, lstm_bhh, bn_g, bn_b, bn_rm, bn_rv, m1_w, m1_b, m2_w,
           m2_b, p_w, p_b):
    f32 = jnp.float32
    src = edge_index[0]
    dst = edge_index[1]
    pad_e = E_PAD - E
    src3 = jnp.concatenate([src, jnp.zeros((pad_e,), jnp.int32)]
                           ).reshape(NW, CPW, CHUNK)
    dst3 = jnp.concatenate([dst, jnp.full((pad_e,), N, jnp.int32)]
                           ).reshape(NW, CPW, CHUNK)
    x_p = jnp.pad(x, ((0, N_PAD - N), (0, 0)))
    ea_p = jnp.pad(edge_attr, ((0, pad_e), (0, 0)))
    zeros32 = jnp.zeros((N_PAD, ATOM), f32)
    zeros48 = jnp.zeros((N_PAD, 48), f32)
    bf16 = jnp.bfloat16
    rmat = jnp.kron(jnp.eye(ATOM, dtype=bf16), jnp.ones((1, ATOM), bf16))
    n2bm = n2_b.reshape(ATOM, ATOM)
    n2wT = n2_w.T.astype(bf16)

    def g3(w):
        return w[0:ATOM].T, w[ATOM:2 * ATOM].T, w[2 * ATOM:3 * ATOM].T

    wir, wiz, win = g3(gru_wih)
    whr, whz, whn = g3(gru_whh)
    row = lambda v: v.reshape(1, -1)
    gw = (wir, wiz, win, whr, whz, whn,
          row(gru_bih[0:ATOM] + gru_bhh[0:ATOM]),
          row(gru_bih[ATOM:2 * ATOM] + gru_bhh[ATOM:2 * ATOM]),
          row(gru_bih[2 * ATOM:]), row(gru_bhh[2 * ATOM:]))

    def g4(w):
        return tuple(w[k * ATOM:(k + 1) * ATOM] for k in range(4))

    wi, wf, wg, wo = g4(lstm_wih)
    ui, uf, ug, uo = g4(lstm_whh)
    col = lambda v: v.reshape(-1, 1)
    lb = tuple(col(lstm_bih[k * ATOM:(k + 1) * ATOM]
                   + lstm_bhh[k * ATOM:(k + 1) * ATOM]) for k in range(4))
    tail_w = (wi, wf, wg, wo, ui, uf, ug, uo) + lb + (
        col(bn_g), col(bn_b), col(bn_rm), col(bn_rv),
        m1_w, col(m1_b), m2_w, col(m2_b), p_w, col(p_b))

    batch2d = jnp.pad(batch, (0, N_PAD - N),
                      constant_values=NGRAPH).reshape(1, N_PAD)
    batchT = batch2d.reshape(N_PAD, 1)

    xh = _linrelu(x_p, lin_w.T, lin_b.reshape(1, -1), 2048)
    hidden = _linrelu(ea_p, n1_w.T, n1_b.reshape(1, -1), 2048,
                      out_dtype=bf16)

    xs = _sc_gather(xh, src3)
    msg48 = _tc_msg(hidden, xs, n2wT, rmat, n2bm, width=ATOM + 16)
    parts48 = _sc_scatter48(msg48, dst3, zeros48)
    h1 = _tc_gru(parts48, xh, gw)

    xs2 = _sc_gather(h1, src3)
    msg2 = _tc_msg(hidden, xs2, n2wT, rmat, n2bm)
    parts32 = _sc_scatter32(msg2, dst3, zeros32)
    out_t = _tc_tail(parts32, parts48, h1, gw, batch2d, batchT, tail_w)
    return out_t.reshape(NGRAPH, 1)


# speedup vs baseline: 1.0348x; 1.0001x over previous
import jax
import jax.numpy as jnp
from jax import lax
from jax.experimental import pallas as pl
from jax.experimental.pallas import tpu as pltpu
from jax.experimental.pallas import tpu_sc as plsc

N = 10000
E = 160000
NODE_DIM = 128
EDGE_DIM = 16
ATOM = 32
CONV = 128
NGRAPH = 64

NC = 2
NS = 16
NW = NC * NS
CHUNK = 128
N_PAD = 10240
E_PAD = 163840
BPW = E_PAD // NW
CPW = BPW // CHUNK
RPT = N_PAD // NS

_SC_MESH = dict(core_axis_name="c", subcore_axis_name="s",
                num_cores=NC, num_subcores=NS)


NB = 4
GQ = 4


def _make_sc_gather(cpw):
    cpq = cpw // GQ
    qrows = cpq * CHUNK
    bpw = cpw * CHUNK

    def body(tab_hbm, idx_hbm, out_hbm, idx_all, rows, *sems):
        gs, ws = sems[:2], sems[2:]
        c = lax.axis_index("c")
        s = lax.axis_index("s")
        wid = c * NS + s
        base = wid * bpw
        pltpu.sync_copy(idx_hbm.at[wid], idx_all)
        gd = [[None] * cpq for _ in range(2)]
        wd = [None, None]

        def issue(qtr):
            b = qtr % 2
            for j in range(cpq):
                jj = qtr * cpq + j
                gd[b][j] = pltpu.async_copy(
                    tab_hbm.at[idx_all.at[jj]],
                    rows.at[b, pl.ds(j * CHUNK, CHUNK)], gs[b])

        issue(0)
        for qtr in range(GQ):
            b = qtr % 2
            if qtr + 1 < GQ:
                if qtr >= 1:
                    wd[(qtr + 1) % 2].wait()
                issue(qtr + 1)
            for j in range(cpq):
                gd[b][j].wait()
            wd[b] = pltpu.async_copy(
                rows.at[b], out_hbm.at[pl.ds(base + qtr * qrows, qrows)],
                ws[b])
        wd[0].wait()
        wd[1].wait()

    def run(tab, idx3):
        return pl.kernel(
            body,
            out_type=jax.ShapeDtypeStruct((NW * bpw, ATOM), jnp.float32),
            compiler_params=pltpu.CompilerParams(use_tc_tiling_on_sc=False),
            mesh=plsc.VectorSubcoreMesh(**_SC_MESH),
            scratch_types=[
                pltpu.VMEM((cpw, CHUNK), jnp.int32),
                pltpu.VMEM((2, qrows, ATOM), jnp.float32),
            ] + [pltpu.SemaphoreType.DMA] * 4,
        )(tab, idx3)

    return run


_sc_gather = _make_sc_gather(CPW)


def _make_sc_scatter(width, cpw=CPW):
    bpw = cpw * CHUNK

    def body(vals_hbm, idx_hbm, zeros_hbm, out_hbm, idx_all, vals, acc, *sems):
        ls, ss = sems[:NB], sems[NB:]
        c = lax.axis_index("c")
        s = lax.axis_index("s")
        wid = c * NS + s
        base = wid * bpw
        pltpu.sync_copy(zeros_hbm.at[pl.ds(s * RPT, RPT)],
                        acc.at[pl.ds(s * RPT, RPT)])
        pltpu.sync_copy(idx_hbm.at[wid], idx_all)
        plsc.subcore_barrier()
        ld = [None] * cpw
        sd = [None] * cpw

        def start(j):
            b = j % NB
            ld[j] = pltpu.async_copy(
                vals_hbm.at[pl.ds(base + j * CHUNK, CHUNK)], vals.at[b], ls[b])

        start(0)
        for j in range(cpw):
            b = j % NB
            if j + 1 < cpw:
                if j + 1 >= NB:
                    sd[j + 1 - NB].wait()
                start(j + 1)
            ld[j].wait()
            sd[j] = pltpu.async_copy(vals.at[b], acc.at[idx_all.at[j]],
                                     ss[b], add=True)
        for j in range(cpw - NB, cpw):
            sd[j].wait()
        plsc.subcore_barrier()
        pltpu.sync_copy(acc.at[pl.ds(s * RPT, RPT)],
                        out_hbm.at[c, pl.ds(s * RPT, RPT)])

    def run(vals, idx3, zeros):
        return pl.kernel(
            body,
            out_type=jax.ShapeDtypeStruct((NC, N_PAD, width), jnp.float32),
            compiler_params=pltpu.CompilerParams(use_tc_tiling_on_sc=False),
            mesh=plsc.VectorSubcoreMesh(**_SC_MESH),
            scratch_types=[
                pltpu.VMEM((cpw, CHUNK), jnp.int32),
                pltpu.VMEM((NB, CHUNK, width), jnp.float32),
                pltpu.VMEM_SHARED((N_PAD, width), jnp.float32),
            ] + [pltpu.SemaphoreType.DMA] * (2 * NB),
        )(vals, idx3, zeros)

    return run


_sc_scatter32 = _make_sc_scatter(ATOM, CPW)
_sc_scatter48 = _make_sc_scatter(48, CPW)


def _linrelu_body(x_ref, w_ref, b_ref, o_ref):
    o_ref[...] = jax.nn.relu(
        jnp.dot(x_ref[...], w_ref[...], preferred_element_type=jnp.float32)
        + b_ref[...]).astype(o_ref.dtype)


def _linrelu(x, w, b, tr, out_dtype=jnp.float32):
    rows, k = x.shape
    d = w.shape[1]
    return pl.pallas_call(
        _linrelu_body,
        grid=(rows // tr,),
        in_specs=[
            pl.BlockSpec((tr, k), lambda i: (i, 0)),
            pl.BlockSpec((k, d), lambda i: (0, 0)),
            pl.BlockSpec((1, d), lambda i: (0, 0)),
        ],
        out_specs=pl.BlockSpec((tr, d), lambda i: (i, 0)),
        out_shape=jax.ShapeDtypeStruct((rows, d), out_dtype),
    )(x, w, b)


TE = 2048


def _msg_body(hid_ref, xs_ref, n2wT_ref, r_ref, n2bm_ref, o_ref):
    hid = hid_ref[...]
    xs = xs_ref[...]
    xsb = xs.astype(jnp.bfloat16)
    q = None
    for j in range(8):
        wj = jnp.dot(hid, n2wT_ref[:, 128 * j:128 * (j + 1)],
                     preferred_element_type=jnp.float32)
        xej = jnp.dot(xsb, r_ref[:, 128 * j:128 * (j + 1)],
                      preferred_element_type=jnp.float32)
        pj = wj * xej
        q = pj if q is None else q + pj
    msg = (lax.slice(q, (0, 0), (TE, 32)) + lax.slice(q, (0, 32), (TE, 64))
           + lax.slice(q, (0, 64), (TE, 96))
           + lax.slice(q, (0, 96), (TE, 128)))
    msg = msg + jnp.dot(xs, n2bm_ref[...], preferred_element_type=jnp.float32)
    if o_ref.shape[1] == ATOM + 16:
        msg = jnp.concatenate(
            [msg, jnp.ones((TE, 16), jnp.float32)], axis=1)
    o_ref[...] = msg


def _tc_msg(hidden, xs, n2wT, rmat, n2bm, width=ATOM, hid_off=0):
    aa = ATOM * ATOM
    rows = xs.shape[0]
    return pl.pallas_call(
        _msg_body,
        grid=(rows // TE,),
        in_specs=[
            pl.BlockSpec((TE, CONV), lambda i, o=hid_off: (i + o, 0)),
            pl.BlockSpec((TE, ATOM), lambda i: (i, 0)),
            pl.BlockSpec((CONV, aa), lambda i: (0, 0)),
            pl.BlockSpec((ATOM, aa), lambda i: (0, 0)),
            pl.BlockSpec((ATOM, ATOM), lambda i: (0, 0)),
        ],
        out_specs=pl.BlockSpec((TE, width), lambda i: (i, 0)),
        out_shape=jax.ShapeDtypeStruct((rows, width), jnp.float32),
    )(hidden, xs, n2wT, rmat, n2bm)


TN = 2048


def _gru_cell_tc(agg, deg, h, gw_refs):
    wir, wiz, win, whr, whz, whn, brz_r, brz_z, bin_, bhn = gw_refs
    m = jax.nn.relu(agg / deg)

    def mm(a, b):
        return jnp.dot(a, b[...], preferred_element_type=jnp.float32)

    r = jax.nn.sigmoid(mm(m, wir) + mm(h, whr) + brz_r[...])
    z = jax.nn.sigmoid(mm(m, wiz) + mm(h, whz) + brz_z[...])
    n = jnp.tanh(mm(m, win) + bin_[...] + r * (mm(h, whn) + bhn[...]))
    return (1.0 - z) * n + z * h


def _gru_body(pa_ref, h_ref, wir, wiz, win, whr, whz, whn,
              brz_r, brz_z, bin_, bhn, o_ref):
    agg = pa_ref[0, :, 0:ATOM] + pa_ref[1, :, 0:ATOM]
    deg = jnp.maximum(
        pa_ref[0, :, ATOM:ATOM + 1] + pa_ref[1, :, ATOM:ATOM + 1], 1.0)
    o_ref[...] = _gru_cell_tc(
        agg, deg, h_ref[...],
        (wir, wiz, win, whr, whz, whn, brz_r, brz_z, bin_, bhn))


def _tc_gru(parts48, h, gw):
    full = lambda shape: pl.BlockSpec(shape, lambda i: tuple(0 for _ in shape))
    return pl.pallas_call(
        _gru_body,
        grid=(N_PAD // TN,),
        in_specs=[
            pl.BlockSpec((NC, TN, ATOM + 16), lambda i: (0, i, 0)),
            pl.BlockSpec((TN, ATOM), lambda i: (i, 0)),
        ] + [full((ATOM, ATOM))] * 6 + [full((1, ATOM))] * 4,
        out_specs=pl.BlockSpec((TN, ATOM), lambda i: (i, 0)),
        out_shape=jax.ShapeDtypeStruct((N_PAD, ATOM), jnp.float32),
    )(parts48, h, *gw)


EMB_STEPS = 3


def _tail_body(p32_ref, p48_ref, h_ref,
               gwir, gwiz, gwin, gwhr, gwhz, gwhn,
               gbrz_r, gbrz_z, gbin, gbhn,
               batch_ref, batchT_ref, wi, wf, wg, wo, ui, uf, ug, uo,
               bi, bf, bg_, bo, bng, bnb, bnrm, bnrv, m1w, m1b, m2w, m2b,
               pw, pb, o_ref):
    agg = p32_ref[0] + p32_ref[1]
    deg = jnp.maximum(
        p48_ref[0, :, ATOM:ATOM + 1] + p48_ref[1, :, ATOM:ATOM + 1], 1.0)
    h2 = _gru_cell_tc(
        agg, deg, h_ref[...],
        (gwir, gwiz, gwin, gwhr, gwhz, gwhn, gbrz_r, gbrz_z, gbin, gbhn))
    xhT = h2.T
    batch = batch_ref[...]
    batchT = batchT_ref[...]
    gids = lax.broadcasted_iota(jnp.int32, (NGRAPH, N_PAD), 0)
    gidsT = lax.broadcasted_iota(jnp.int32, (N_PAD, NGRAPH), 1)
    mb = gids == batch
    mf = mb.astype(jnp.float32)
    mfT = (gidsT == batchT).astype(jnp.float32)

    def mm(a, b):
        return jnp.dot(a, b, preferred_element_type=jnp.float32)

    qsT = jnp.zeros((2 * ATOM, NGRAPH), jnp.float32)
    hsT = jnp.zeros((ATOM, NGRAPH), jnp.float32)
    csT = jnp.zeros((ATOM, NGRAPH), jnp.float32)
    for _ in range(EMB_STEPS):
        ig = jax.nn.sigmoid(mm(wi[...], qsT) + mm(ui[...], hsT) + bi[...])
        fg = jax.nn.sigmoid(mm(wf[...], qsT) + mm(uf[...], hsT) + bf[...])
        gg = jnp.tanh(mm(wg[...], qsT) + mm(ug[...], hsT) + bg_[...])
        og = jax.nn.sigmoid(mm(wo[...], qsT) + mm(uo[...], hsT) + bo[...])
        csT = fg * csT + ig * gg
        hsT = og * jnp.tanh(csT)
        qT = hsT
        qbT = mm(qT, mf)
        e = jnp.sum(xhT * qbT, axis=0, keepdims=True)
        e_b = jnp.broadcast_to(e, (NGRAPH, N_PAD))
        mmax = jnp.max(jnp.where(mb, e_b, -1e30), axis=1, keepdims=True)
        mmax_n = jnp.sum(mf * mmax, axis=0, keepdims=True)
        a = jnp.exp(e - mmax_n)
        denom = jnp.sum(mf * a, axis=1, keepdims=True)
        denom_n = jnp.sum(mf * denom, axis=0, keepdims=True)
        anorm = jnp.where(denom_n > 0.0,
                          a / jnp.maximum(denom_n, 1e-30), 0.0)
        rT = mm(xhT * anorm, mfT)
        qsT = jnp.concatenate([qT, rT], axis=0)

    o = (qsT - bnrm[...]) / jnp.sqrt(bnrv[...] + 1e-5) * bng[...] + bnb[...]
    o1 = jax.nn.relu(mm(m1w[...], o) + m1b[...])
    o2 = jax.nn.relu(mm(m2w[...], o1) + m2b[...])
    o_ref[...] = mm(pw[...], o2) + pb[...]


def _tc_tail(p32, p48, h, gw, batch2d, batchT, weights):
    return pl.pallas_call(
        _tail_body,
        out_shape=jax.ShapeDtypeStruct((1, NGRAPH), jnp.float32),
    )(p32, p48, h, *gw, batch2d, batchT, *weights)


def kernel(x, edge_attr, edge_index, batch, lin_w, lin_b, n1_w, n1_b, n2_w,
           n2_b, gru_wih, gru_whh, gru_bih, gru_bhh, lstm_wih, lstm_whh,
           lstm_bih, lstm_bhh, bn_g, bn_b, bn_rm, bn_rv, m1_w, m1_b, m2_w,
           m2_b, p_w, p_b):
    f32 = jnp.float32
    src = edge_index[0]
    dst = edge_index[1]
    pad_e = E_PAD - E
    src3 = jnp.concatenate([src, jnp.zeros((pad_e,), jnp.int32)]
                           ).reshape(NW, CPW, CHUNK)
    dst3 = jnp.concatenate([dst, jnp.full((pad_e,), N, jnp.int32)]
                           ).reshape(NW, CPW, CHUNK)
    x_p = jnp.pad(x, ((0, N_PAD - N), (0, 0)))
    ea_p = jnp.pad(edge_attr, ((0, pad_e), (0, 0)))
    zeros32 = jnp.zeros((N_PAD, ATOM), f32)
    zeros48 = jnp.zeros((N_PAD, 48), f32)
    bf16 = jnp.bfloat16
    rmat = jnp.kron(jnp.eye(ATOM, dtype=bf16), jnp.ones((1, ATOM), bf16))
    n2bm = n2_b.reshape(ATOM, ATOM)
    n2wT = n2_w.T.astype(bf16)

    def g3(w):
        return w[0:ATOM].T, w[ATOM:2 * ATOM].T, w[2 * ATOM:3 * ATOM].T

    wir, wiz, win = g3(gru_wih)
    whr, whz, whn = g3(gru_whh)
    row = lambda v: v.reshape(1, -1)
    gw = (wir, wiz, win, whr, whz, whn,
          row(gru_bih[0:ATOM] + gru_bhh[0:ATOM]),
          row(gru_bih[ATOM:2 * ATOM] + gru_bhh[ATOM:2 * ATOM]),
          row(gru_bih[2 * ATOM:]), row(gru_bhh[2 * ATOM:]))

    def g4(w):
        return tuple(w[k * ATOM:(k + 1) * ATOM] for k in range(4))

    wi, wf, wg, wo = g4(lstm_wih)
    ui, uf, ug, uo = g4(lstm_whh)
    col = lambda v: v.reshape(-1, 1)
    lb = tuple(col(lstm_bih[k * ATOM:(k + 1) * ATOM]
                   + lstm_bhh[k * ATOM:(k + 1) * ATOM]) for k in range(4))
    tail_w = (wi, wf, wg, wo, ui, uf, ug, uo) + lb + (
        col(bn_g), col(bn_b), col(bn_rm), col(bn_rv),
        m1_w, col(m1_b), m2_w, col(m2_b), p_w, col(p_b))

    batch2d = jnp.pad(batch, (0, N_PAD - N),
                      constant_values=NGRAPH).reshape(1, N_PAD)
    batchT = batch2d.reshape(N_PAD, 1)

    xh = _linrelu(x_p, lin_w.T, lin_b.reshape(1, -1), 2048)
    hidden = _linrelu(ea_p, n1_w.T, n1_b.reshape(1, -1), 2048,
                      out_dtype=bf16)

    xs = _sc_gather(xh, src3)
    msg48 = _tc_msg(hidden, xs, n2wT, rmat, n2bm, width=ATOM + 16)
    parts48 = _sc_scatter48(msg48, dst3, zeros48)
    h1 = _tc_gru(parts48, xh, gw)

    xs2 = _sc_gather(h1, src3)
    msg2 = _tc_msg(hidden, xs2, n2wT, rmat, n2bm)
    parts32 = _sc_scatter32(msg2, dst3, zeros32)
    out_t = _tc_tail(parts32, parts48, h1, gw, batch2d, batchT, tail_w)
    return out_t.reshape(NGRAPH, 1)
